# speedup vs baseline: 1.9768x; 1.9768x over previous
import functools

import jax
import jax.numpy as jnp
from jax.experimental import pallas as pl

B = 16
N = 128
HID = 64
K = 128


def _tc_body(mask_ref, g_ref, nodes_ref, wg_ref, wa_ref, wb_ref,
             aw1_ref, ab1_ref, aw2_ref, ab2_ref, ab0_ref,
             cw0_ref, cb0_ref, cw1_ref, cb1_ref, cw2_ref, cb2_ref,
             pi_ref, val_ref):
    m2 = mask_ref[0]
    bits = jax.lax.bitcast_convert_type(m2, jnp.int32)
    u = jnp.where(bits < 0, ~bits, bits ^ jnp.int32(-2147483648)).astype(jnp.uint32)

    def bs_body(k, t):
        cand = t | (jnp.uint32(1) << (31 - k).astype(jnp.uint32))
        cnt = jnp.sum((u >= cand).astype(jnp.int32))
        return jnp.where(cnt >= K, cand, t)

    tv = jax.lax.fori_loop(0, 32, bs_body, jnp.uint32(0))

    gt = u > tv
    eq = u == tv
    cnt_gt = jnp.sum(gt.astype(jnp.int32))
    m_f = (K - cnt_gt).astype(jnp.float32)

    eqf = eq.astype(jnp.float32)
    r_i = jax.lax.broadcasted_iota(jnp.int32, (N, N), 0)
    c_i = jax.lax.broadcasted_iota(jnp.int32, (N, N), 1)
    su = (r_i < c_i).astype(jnp.float32)
    sl = (c_i < r_i).astype(jnp.float32)
    row_prefix = jnp.dot(eqf, su, preferred_element_type=jnp.float32)
    rowsum = jnp.sum(eqf, axis=1, keepdims=True)
    row_offs = jnp.dot(sl, rowsum, preferred_element_type=jnp.float32)
    pe = row_prefix + row_offs
    accept = gt | (eq & (pe < m_f))

    nodes = nodes_ref[0]
    g = g_ref[0]
    a = jnp.dot(nodes, wa_ref[...], preferred_element_type=jnp.float32)
    b = jnp.dot(nodes, wb_ref[...], preferred_element_type=jnp.float32)
    c = jnp.dot(g, wg_ref[...], preferred_element_type=jnp.float32) + ab0_ref[...]
    h0 = jax.nn.relu(b[:, None, :] + (a + c)[None, :, :])
    h0 = h0.reshape(N * N, HID)
    h1 = jax.nn.relu(jnp.dot(h0, aw1_ref[...], preferred_element_type=jnp.float32)
                     + ab1_ref[...])
    lg = jnp.dot(h1, aw2_ref[...], preferred_element_type=jnp.float32) + ab2_ref[...]
    lg2 = lg.reshape(N, N)

    neg = jnp.float32(-1e30)
    lmask = jnp.where(accept, lg2, neg)
    mx = jnp.max(lmask)
    e = jnp.where(accept, jnp.exp(lg2 - mx), 0.0)
    pi_ref[0] = e / jnp.sum(e)

    hv = jax.nn.relu(jnp.dot(g, cw0_ref[...], preferred_element_type=jnp.float32)
                     + cb0_ref[...])
    hv = jax.nn.relu(jnp.dot(hv, cw1_ref[...], preferred_element_type=jnp.float32)
                     + cb1_ref[...])
    val_ref[0] = jnp.dot(hv, cw2_ref[...], preferred_element_type=jnp.float32) + cb2_ref[...]


@jax.jit
def kernel(embedded_features, aW0, ab0, aW1, ab1, aW2, ab2,
           cW0, cb0, cW1, cb1, cW2, cb2):
    gan = embedded_features[:, :, :HID]
    g3 = gan[:, :1, :]
    nodes = gan[:, 1:, :]
    maskf = embedded_features[:, 1:, HID:]

    wg = aW0[:HID]
    wa = aW0[HID:2 * HID]
    wb = aW0[2 * HID:]
    ab0r = ab0.reshape(1, HID)
    ab1r = ab1.reshape(1, HID)
    ab2r = ab2.reshape(1, 1)
    cb0r = cb0.reshape(1, HID)
    cb1r = cb1.reshape(1, HID)
    cb2r = cb2.reshape(1, 1)

    full = lambda shape: pl.BlockSpec(shape, lambda i: (0,) * len(shape))
    per_b3 = lambda s1, s2: pl.BlockSpec((1, s1, s2), lambda i: (i, 0, 0))

    pi, value = pl.pallas_call(
        _tc_body,
        grid=(B,),
        in_specs=[
            per_b3(N, N),
            per_b3(1, HID),
            per_b3(N, HID),
            full((HID, HID)),
            full((HID, HID)),
            full((HID, HID)),
            full((HID, HID)),
            full((1, HID)),
            full((HID, 1)),
            full((1, 1)),
            full((1, HID)),
            full((HID, HID)),
            full((1, HID)),
            full((HID, HID)),
            full((1, HID)),
            full((HID, 1)),
            full((1, 1)),
        ],
        out_specs=[per_b3(N, N), per_b3(1, 1)],
        out_shape=[
            jax.ShapeDtypeStruct((B, N, N), jnp.float32),
            jax.ShapeDtypeStruct((B, 1, 1), jnp.float32),
        ],
    )(maskf, g3, nodes, wg, wa, wb, aW1, ab1r, aW2, ab2r, ab0r,
      cW0, cb0r, cW1, cb1r, cW2, cb2r)

    return pi.reshape(B, N * N), value


# speedup vs baseline: 3.6186x; 1.8306x over previous
import functools

import numpy as np

import jax
import jax.numpy as jnp
from jax import lax
from jax.experimental import pallas as pl
from jax.experimental.pallas import tpu as pltpu
from jax.experimental.pallas import tpu_sc as plsc

B = 16
N = 128
HID = 64
K = 128
ROW_N = N * N
L = 16
NV = ROW_N // L
SEG = ROW_N // L
HI_CAP = 128
MIN32 = np.int32(-2147483648)


def _iota16():
    return lax.broadcasted_iota(jnp.int32, (L,), 0)


def _crossing(hist_ref, kr):
    lanes = _iota16()
    tgs = []
    sums = []
    for g in range(16):
        tg = jnp.zeros((L,), jnp.int32)
        for m in range(L):
            tg = tg + hist_ref[pl.ds(m * 256 + g * L, L)]
        tgs.append(tg)
        sums.append(jnp.sum(tg))
    bstar = jnp.int32(0)
    sstar = jnp.int32(0)
    above = jnp.int32(0)
    for g in range(15, -1, -1):
        tg = tgs[g]
        sfx_incl = lax.rev(plsc.cumsum(lax.rev(tg, (0,))), (0,))
        s_b = above + sfx_incl - tg
        okv = (s_b < kr) & (s_b + tg >= kr)
        bstar = bstar + jnp.sum(jnp.where(okv, g * L + lanes, 0))
        sstar = sstar + jnp.sum(jnp.where(okv, s_b, 0))
        above = above + sums[g]
    return bstar, sstar


def _sc_topk_body(mask_hbm, out_hbm, rbuf, hist, hibuf, cu, ci,
                  ccu, cci, outbuf):
    wid = lax.axis_index("s") * 2 + lax.axis_index("c")

    @pl.when(wid < B)
    def _():
        lanes = _iota16()
        ones16 = jnp.ones((L,), jnp.int32)
        zero16 = jnp.zeros((L,), jnp.int32)
        pltpu.sync_copy(mask_hbm.at[wid], rbuf)

        def zh(i, _):
            hist[pl.ds(i * L, L)] = zero16
            return 0

        lax.fori_loop(0, 256 * L // L, zh, 0, unroll=4)

        def s1(i, _):
            bits = rbuf[pl.ds(i * L, L)]
            r = jnp.where(bits < 0, ~bits, bits ^ MIN32)
            rbuf[pl.ds(i * L, L)] = r
            b0 = lax.shift_right_logical(r, 24)
            plsc.addupdate_scatter(hist, [lanes * 256 + b0], ones16)
            return 0

        lax.fori_loop(0, NV, s1, 0, unroll=4)

        b0star, sstar0 = _crossing(hist, jnp.int32(K))
        kr = K - sstar0

        base_idx = lanes * SEG

        def s2(t, carry):
            hioffs, coffs = carry
            gidx = base_idx + t
            r = plsc.load_gather(rbuf, [gidx])
            byte0 = lax.shift_right_logical(r, 24)
            hi = byte0 > b0star
            cd = byte0 == b0star
            plsc.store_scatter(hibuf, [hioffs], gidx, mask=hi)
            plsc.store_scatter(cu, [coffs], r, mask=cd)
            plsc.store_scatter(ci, [coffs], gidx, mask=cd)
            return (hioffs + hi.astype(jnp.int32), coffs + cd.astype(jnp.int32))

        hioffs, coffs = lax.fori_loop(
            0, SEG, s2, (lanes * HI_CAP, lanes * SEG), unroll=4)
        ccnts = coffs - lanes * SEG
        hcnts = hioffs - lanes * HI_CAP

        hptr = jnp.int32(0)
        for l in range(L):
            c_l = hcnts[l]

            def cph2(tt, _, l=l):
                v = hibuf[pl.ds(l * HI_CAP + tt * L, L)]
                plsc.store_scatter(outbuf, [hptr + tt * L + lanes], v)
                return 0

            lax.fori_loop(0, (c_l + L - 1) // L, cph2, 0)
            hptr = hptr + c_l

        cptr = jnp.int32(0)
        for l in range(L):
            c_l = ccnts[l]

            def cpc(tt, _, l=l):
                vr = cu[pl.ds(l * SEG + tt * L, L)]
                vi = ci[pl.ds(l * SEG + tt * L, L)]
                dst = cptr + tt * L + lanes
                plsc.store_scatter(ccu, [dst], vr)
                plsc.store_scatter(cci, [dst], vi)
                return 0

            lax.fori_loop(0, (c_l + L - 1) // L, cpc, 0)
            cptr = cptr + c_l
        cn = cptr

        pfx = b0star
        for sh in (16, 8, 0):
            lax.fori_loop(0, 256, zh, 0, unroll=4)
            nv = (cn + L - 1) // L

            def sp(tt, _, sh=sh, pfx=pfx):
                addr = tt * L + lanes
                valid = addr < cn
                r = ccu[pl.ds(tt * L, L)]
                act = valid & (lax.shift_right_logical(r, sh + 8) == pfx)
                b = lax.shift_right_logical(r, sh) & 255
                plsc.addupdate_scatter(hist, [lanes * 256 + b], ones16, mask=act)
                return 0

            lax.fori_loop(0, nv, sp, 0)
            bstar, sstar = _crossing(hist, kr)
            kr = kr - sstar
            pfx = lax.shift_left(pfx, 8) | bstar
        tv = pfx
        tv_s = tv ^ MIN32

        nv = (cn + L - 1) // L

        def fs(tt, carry):
            ptr, neq = carry
            addr = tt * L + lanes
            valid = addr < cn
            r = ccu[pl.ds(tt * L, L)]
            iv = cci[pl.ds(tt * L, L)]
            rs = r ^ MIN32
            gt = valid & (rs > tv_s)
            eq = valid & (r == tv)
            eqc = eq.astype(jnp.int32)
            inc = plsc.cumsum(eqc)
            take = eq & ((inc - eqc + neq) < kr)
            acc = gt | take
            acci = acc.astype(jnp.int32)
            inca = plsc.cumsum(acci)
            plsc.store_scatter(outbuf, [ptr + inca - acci], iv, mask=acc)
            return (ptr + jnp.sum(acci), neq + jnp.sum(eqc))

        lax.fori_loop(0, nv, fs, (hptr, jnp.int32(0)))
        pltpu.sync_copy(outbuf.at[pl.ds(0, K)], out_hbm.at[wid])


_sc_topk = functools.partial(
    pl.kernel,
    out_type=jax.ShapeDtypeStruct((B, K), jnp.int32),
    mesh=plsc.VectorSubcoreMesh(core_axis_name="c", subcore_axis_name="s",
                                num_cores=2, num_subcores=16),
    compiler_params=pltpu.CompilerParams(needs_layout_passes=False),
    scratch_types=[
        pltpu.VMEM((ROW_N,), jnp.int32),
        pltpu.VMEM((256 * L,), jnp.int32),
        pltpu.VMEM((HI_CAP * L,), jnp.int32),
        pltpu.VMEM((ROW_N,), jnp.int32),
        pltpu.VMEM((ROW_N,), jnp.int32),
        pltpu.VMEM((ROW_N + L,), jnp.int32),
        pltpu.VMEM((ROW_N + L,), jnp.int32),
        pltpu.VMEM((K + L,), jnp.int32),
    ],
)(_sc_topk_body)


def _tc_body(idx_ref, g_ref, nodes_ref, wg_ref, wa_ref, wb_ref,
             aw1_ref, ab1_ref, aw2_ref, ab2_ref, ab0_ref,
             cw0_ref, cb0_ref, cw1_ref, cb1_ref, cw2_ref, cb2_ref,
             pi_ref, val_ref):
    idxv = idx_ref[0]
    i_sel = lax.shift_right_logical(idxv, 7)
    j_sel = idxv & 127
    r_i = lax.broadcasted_iota(jnp.int32, (N, N), 0)
    ohit = (r_i == i_sel).astype(jnp.float32)
    ohjt = (r_i == j_sel).astype(jnp.float32)

    nodes = nodes_ref[0]
    g = g_ref[0]
    a = jnp.dot(nodes, wa_ref[...], preferred_element_type=jnp.float32)
    bm = jnp.dot(nodes, wb_ref[...], preferred_element_type=jnp.float32)
    c = jnp.dot(g, wg_ref[...], preferred_element_type=jnp.float32) + ab0_ref[...]

    ohj = jnp.transpose(ohjt)
    ohi = jnp.transpose(ohit)
    hp = jax.lax.Precision.HIGHEST
    a_sel = jnp.dot(ohj, a, preferred_element_type=jnp.float32, precision=hp)
    b_sel = jnp.dot(ohi, bm, preferred_element_type=jnp.float32, precision=hp)
    h0 = jax.nn.relu(a_sel + b_sel + c)
    h1 = jax.nn.relu(jnp.dot(h0, aw1_ref[...], preferred_element_type=jnp.float32)
                     + ab1_ref[...])
    lg = jnp.dot(h1, aw2_ref[...], preferred_element_type=jnp.float32) + ab2_ref[...]

    mx = jnp.max(lg)
    e = jnp.exp(lg - mx)
    pi_col = e / jnp.sum(e)

    ohj_scaled = ohj * pi_col
    pi_ref[0] = jnp.dot(ohit, ohj_scaled, preferred_element_type=jnp.float32,
                        precision=hp)

    hv = jax.nn.relu(jnp.dot(g, cw0_ref[...], preferred_element_type=jnp.float32)
                     + cb0_ref[...])
    hv = jax.nn.relu(jnp.dot(hv, cw1_ref[...], preferred_element_type=jnp.float32)
                     + cb1_ref[...])
    val_ref[0] = jnp.dot(hv, cw2_ref[...], preferred_element_type=jnp.float32) + cb2_ref[...]


@jax.jit
def kernel(embedded_features, aW0, ab0, aW1, ab1, aW2, ab2,
           cW0, cb0, cW1, cb1, cW2, cb2):
    gan = embedded_features[:, :, :HID]
    g3 = gan[:, :1, :]
    nodes = gan[:, 1:, :]
    maskrow = embedded_features[:, 1:, HID:].reshape(B, ROW_N)
    maskbits = lax.bitcast_convert_type(maskrow, jnp.int32)

    idx = _sc_topk(maskbits)
    idx3 = idx.reshape(B, 1, K)

    wg = aW0[:HID]
    wa = aW0[HID:2 * HID]
    wb = aW0[2 * HID:]
    ab0r = ab0.reshape(1, HID)
    ab1r = ab1.reshape(1, HID)
    ab2r = ab2.reshape(1, 1)
    cb0r = cb0.reshape(1, HID)
    cb1r = cb1.reshape(1, HID)
    cb2r = cb2.reshape(1, 1)

    full = lambda shape: pl.BlockSpec(shape, lambda i: (0,) * len(shape))
    per_b3 = lambda s1, s2: pl.BlockSpec((1, s1, s2), lambda i: (i, 0, 0))

    pi, value = pl.pallas_call(
        _tc_body,
        grid=(B,),
        in_specs=[
            per_b3(1, K),
            per_b3(1, HID),
            per_b3(N, HID),
            full((HID, HID)),
            full((HID, HID)),
            full((HID, HID)),
            full((HID, HID)),
            full((1, HID)),
            full((HID, 1)),
            full((1, 1)),
            full((1, HID)),
            full((HID, HID)),
            full((1, HID)),
            full((HID, HID)),
            full((1, HID)),
            full((HID, 1)),
            full((1, 1)),
        ],
        out_specs=[per_b3(N, N), per_b3(1, 1)],
        out_shape=[
            jax.ShapeDtypeStruct((B, N, N), jnp.float32),
            jax.ShapeDtypeStruct((B, 1, 1), jnp.float32),
        ],
    )(idx3, g3, nodes, wg, wa, wb, aW1, ab1r, aW2, ab2r, ab0r,
      cW0, cb0r, cW1, cb1r, cW2, cb2r)

    return pi.reshape(B, N * N), value


# speedup vs baseline: 3.6216x; 1.0008x over previous
import functools

import numpy as np

import jax
import jax.numpy as jnp
from jax import lax
from jax.experimental import pallas as pl
from jax.experimental.pallas import tpu as pltpu
from jax.experimental.pallas import tpu_sc as plsc

B = 16
N = 128
HID = 64
K = 128
ROW_N = N * N
L = 16
NV = ROW_N // L
SEG = ROW_N // L
HI_CAP = 128
MIN32 = np.int32(-2147483648)


def _iota16():
    return lax.broadcasted_iota(jnp.int32, (L,), 0)


def _crossing(hist_ref, kr):
    lanes = _iota16()
    tgs = []
    sums = []
    for g in range(16):
        tg = jnp.zeros((L,), jnp.int32)
        for m in range(L):
            tg = tg + hist_ref[pl.ds(m * 256 + g * L, L)]
        tgs.append(tg)
        sums.append(jnp.sum(tg))
    bstar = jnp.int32(0)
    sstar = jnp.int32(0)
    above = jnp.int32(0)
    for g in range(15, -1, -1):
        tg = tgs[g]
        sfx_incl = lax.rev(plsc.cumsum(lax.rev(tg, (0,))), (0,))
        s_b = above + sfx_incl - tg
        okv = (s_b < kr) & (s_b + tg >= kr)
        bstar = bstar + jnp.sum(jnp.where(okv, g * L + lanes, 0))
        sstar = sstar + jnp.sum(jnp.where(okv, s_b, 0))
        above = above + sums[g]
    return bstar, sstar


def _sc_topk_body(mask_hbm, out_hbm, rbuf, hist, hibuf, cu, ci,
                  ccu, cci, outbuf):
    wid = lax.axis_index("s") * 2 + lax.axis_index("c")

    @pl.when(wid < B)
    def _():
        lanes = _iota16()
        ones16 = jnp.ones((L,), jnp.int32)
        zero16 = jnp.zeros((L,), jnp.int32)
        pltpu.sync_copy(mask_hbm.at[wid], rbuf)

        def zh(i, _):
            hist[pl.ds(i * L, L)] = zero16
            return 0

        with jax.named_scope("ph_zero"):
            lax.fori_loop(0, 256 * L // L, zh, 0, unroll=4)

        def s1(i, _):
            bits = rbuf[pl.ds(i * L, L)]
            r = jnp.where(bits < 0, ~bits, bits ^ MIN32)
            rbuf[pl.ds(i * L, L)] = r
            b0 = lax.shift_right_logical(r, 24)
            plsc.addupdate_scatter(hist, [lanes * 256 + b0], ones16)
            return 0

        with jax.named_scope("ph_scan1"):
            lax.fori_loop(0, NV, s1, 0, unroll=4)

        with jax.named_scope("ph_cross0"):
            b0star, sstar0 = _crossing(hist, jnp.int32(K))
        kr = K - sstar0

        base_idx = lanes * SEG

        def s2(t, carry):
            hioffs, coffs = carry
            gidx = base_idx + t
            r = plsc.load_gather(rbuf, [gidx])
            byte0 = lax.shift_right_logical(r, 24)
            hi = byte0 > b0star
            cd = byte0 == b0star
            plsc.store_scatter(hibuf, [hioffs], gidx, mask=hi)
            plsc.store_scatter(cu, [coffs], r, mask=cd)
            plsc.store_scatter(ci, [coffs], gidx, mask=cd)
            return (hioffs + hi.astype(jnp.int32), coffs + cd.astype(jnp.int32))

        with jax.named_scope("ph_scan2"):
            hioffs, coffs = lax.fori_loop(
                0, SEG, s2, (lanes * HI_CAP, lanes * SEG), unroll=4)
        ccnts = coffs - lanes * SEG
        hcnts = hioffs - lanes * HI_CAP

        hptr = jnp.int32(0)
        with jax.named_scope("ph_compact"):
          for l in range(L):
            c_l = hcnts[l]

            def cph2(tt, _, l=l):
                v = hibuf[pl.ds(l * HI_CAP + tt * L, L)]
                plsc.store_scatter(outbuf, [hptr + tt * L + lanes], v)
                return 0

            lax.fori_loop(0, (c_l + L - 1) // L, cph2, 0)
            hptr = hptr + c_l

          cptr = jnp.int32(0)
          for l in range(L):
            c_l = ccnts[l]

            def cpc(tt, _, l=l):
                vr = cu[pl.ds(l * SEG + tt * L, L)]
                vi = ci[pl.ds(l * SEG + tt * L, L)]
                dst = cptr + tt * L + lanes
                plsc.store_scatter(ccu, [dst], vr)
                plsc.store_scatter(cci, [dst], vi)
                return 0

            lax.fori_loop(0, (c_l + L - 1) // L, cpc, 0)
            cptr = cptr + c_l
        cn = cptr

        pfx = b0star
        with jax.named_scope("ph_bytepasses"):
          for sh in (16, 8, 0):
            lax.fori_loop(0, 256, zh, 0, unroll=4)
            nv = (cn + L - 1) // L

            def sp(tt, _, sh=sh, pfx=pfx):
                addr = tt * L + lanes
                valid = addr < cn
                r = ccu[pl.ds(tt * L, L)]
                act = valid & (lax.shift_right_logical(r, sh + 8) == pfx)
                b = lax.shift_right_logical(r, sh) & 255
                plsc.addupdate_scatter(hist, [lanes * 256 + b], ones16, mask=act)
                return 0

            lax.fori_loop(0, nv, sp, 0)
            bstar, sstar = _crossing(hist, kr)
            kr = kr - sstar
            pfx = lax.shift_left(pfx, 8) | bstar
          tv = pfx
        tv_s = tv ^ MIN32

        nv = (cn + L - 1) // L

        def fs(tt, carry):
            ptr, neq = carry
            addr = tt * L + lanes
            valid = addr < cn
            r = ccu[pl.ds(tt * L, L)]
            iv = cci[pl.ds(tt * L, L)]
            rs = r ^ MIN32
            gt = valid & (rs > tv_s)
            eq = valid & (r == tv)
            eqc = eq.astype(jnp.int32)
            inc = plsc.cumsum(eqc)
            take = eq & ((inc - eqc + neq) < kr)
            acc = gt | take
            acci = acc.astype(jnp.int32)
            inca = plsc.cumsum(acci)
            plsc.store_scatter(outbuf, [ptr + inca - acci], iv, mask=acc)
            return (ptr + jnp.sum(acci), neq + jnp.sum(eqc))

        with jax.named_scope("ph_select"):
            lax.fori_loop(0, nv, fs, (hptr, jnp.int32(0)))
        pltpu.sync_copy(outbuf.at[pl.ds(0, K)], out_hbm.at[wid])


_sc_topk = functools.partial(
    pl.kernel,
    out_type=jax.ShapeDtypeStruct((B, K), jnp.int32),
    mesh=plsc.VectorSubcoreMesh(core_axis_name="c", subcore_axis_name="s",
                                num_cores=2, num_subcores=16),
    compiler_params=pltpu.CompilerParams(needs_layout_passes=False),
    scratch_types=[
        pltpu.VMEM((ROW_N,), jnp.int32),
        pltpu.VMEM((256 * L,), jnp.int32),
        pltpu.VMEM((HI_CAP * L,), jnp.int32),
        pltpu.VMEM((ROW_N,), jnp.int32),
        pltpu.VMEM((ROW_N,), jnp.int32),
        pltpu.VMEM((ROW_N + L,), jnp.int32),
        pltpu.VMEM((ROW_N + L,), jnp.int32),
        pltpu.VMEM((K + L,), jnp.int32),
    ],
)(_sc_topk_body)


def _tc_body(idx_ref, g_ref, nodes_ref, wg_ref, wa_ref, wb_ref,
             aw1_ref, ab1_ref, aw2_ref, ab2_ref, ab0_ref,
             cw0_ref, cb0_ref, cw1_ref, cb1_ref, cw2_ref, cb2_ref,
             pi_ref, val_ref):
    idxv = idx_ref[0]
    i_sel = lax.shift_right_logical(idxv, 7)
    j_sel = idxv & 127
    r_i = lax.broadcasted_iota(jnp.int32, (N, N), 0)
    ohit = (r_i == i_sel).astype(jnp.float32)
    ohjt = (r_i == j_sel).astype(jnp.float32)

    nodes = nodes_ref[0]
    g = g_ref[0]
    a = jnp.dot(nodes, wa_ref[...], preferred_element_type=jnp.float32)
    bm = jnp.dot(nodes, wb_ref[...], preferred_element_type=jnp.float32)
    c = jnp.dot(g, wg_ref[...], preferred_element_type=jnp.float32) + ab0_ref[...]

    ohj = jnp.transpose(ohjt)
    ohi = jnp.transpose(ohit)
    hp = jax.lax.Precision.HIGHEST
    a_sel = jnp.dot(ohj, a, preferred_element_type=jnp.float32, precision=hp)
    b_sel = jnp.dot(ohi, bm, preferred_element_type=jnp.float32, precision=hp)
    h0 = jax.nn.relu(a_sel + b_sel + c)
    h1 = jax.nn.relu(jnp.dot(h0, aw1_ref[...], preferred_element_type=jnp.float32)
                     + ab1_ref[...])
    lg = jnp.dot(h1, aw2_ref[...], preferred_element_type=jnp.float32) + ab2_ref[...]

    mx = jnp.max(lg)
    e = jnp.exp(lg - mx)
    pi_col = e / jnp.sum(e)

    ohj_scaled = ohj * pi_col
    pi_ref[0] = jnp.dot(ohit, ohj_scaled, preferred_element_type=jnp.float32,
                        precision=hp)

    hv = jax.nn.relu(jnp.dot(g, cw0_ref[...], preferred_element_type=jnp.float32)
                     + cb0_ref[...])
    hv = jax.nn.relu(jnp.dot(hv, cw1_ref[...], preferred_element_type=jnp.float32)
                     + cb1_ref[...])
    val_ref[0] = jnp.dot(hv, cw2_ref[...], preferred_element_type=jnp.float32) + cb2_ref[...]


@jax.jit
def kernel(embedded_features, aW0, ab0, aW1, ab1, aW2, ab2,
           cW0, cb0, cW1, cb1, cW2, cb2):
    gan = embedded_features[:, :, :HID]
    g3 = gan[:, :1, :]
    nodes = gan[:, 1:, :]
    maskrow = embedded_features[:, 1:, HID:].reshape(B, ROW_N)
    maskbits = lax.bitcast_convert_type(maskrow, jnp.int32)

    idx = _sc_topk(maskbits)
    idx3 = idx.reshape(B, 1, K)

    wg = aW0[:HID]
    wa = aW0[HID:2 * HID]
    wb = aW0[2 * HID:]
    ab0r = ab0.reshape(1, HID)
    ab1r = ab1.reshape(1, HID)
    ab2r = ab2.reshape(1, 1)
    cb0r = cb0.reshape(1, HID)
    cb1r = cb1.reshape(1, HID)
    cb2r = cb2.reshape(1, 1)

    full = lambda shape: pl.BlockSpec(shape, lambda i: (0,) * len(shape))
    per_b3 = lambda s1, s2: pl.BlockSpec((1, s1, s2), lambda i: (i, 0, 0))

    pi, value = pl.pallas_call(
        _tc_body,
        grid=(B,),
        in_specs=[
            per_b3(1, K),
            per_b3(1, HID),
            per_b3(N, HID),
            full((HID, HID)),
            full((HID, HID)),
            full((HID, HID)),
            full((HID, HID)),
            full((1, HID)),
            full((HID, 1)),
            full((1, 1)),
            full((1, HID)),
            full((HID, HID)),
            full((1, HID)),
            full((HID, HID)),
            full((1, HID)),
            full((HID, 1)),
            full((1, 1)),
        ],
        out_specs=[per_b3(N, N), per_b3(1, 1)],
        out_shape=[
            jax.ShapeDtypeStruct((B, N, N), jnp.float32),
            jax.ShapeDtypeStruct((B, 1, 1), jnp.float32),
        ],
    )(idx3, g3, nodes, wg, wa, wb, aW1, ab1r, aW2, ab2r, ab0r,
      cW0, cb0r, cW1, cb1r, cW2, cb2r)

    return pi.reshape(B, N * N), value


# speedup vs baseline: 4.2042x; 1.1609x over previous
import functools

import numpy as np

import jax
import jax.numpy as jnp
from jax import lax
from jax.experimental import pallas as pl
from jax.experimental.pallas import tpu as pltpu
from jax.experimental.pallas import tpu_sc as plsc

B = 16
N = 128
HID = 64
K = 128
ROW_N = N * N
L = 16
HROW = ROW_N // 2
SEG = HROW // L
CCAP = ROW_N + L
MIN32 = np.int32(-2147483648)


def _iota16():
    return lax.broadcasted_iota(jnp.int32, (L,), 0)


def _crossing_from(load_tg, kr):
    lanes = _iota16()
    tgs = []
    sums = []
    for g in range(16):
        tg = load_tg(g)
        tgs.append(tg)
        sums.append(jnp.sum(tg))
    bstar = jnp.int32(0)
    sstar = jnp.int32(0)
    above = jnp.int32(0)
    for g in range(15, -1, -1):
        tg = tgs[g]
        sfx_incl = lax.rev(plsc.cumsum(lax.rev(tg, (0,))), (0,))
        s_b = above + sfx_incl - tg
        okv = (s_b < kr) & (s_b + tg >= kr)
        bstar = bstar + jnp.sum(jnp.where(okv, g * L + lanes, 0))
        sstar = sstar + jnp.sum(jnp.where(okv, s_b, 0))
        above = above + sums[g]
    return bstar, sstar


def _sc_topk_body(mask_hbm, out_hbm, rbuf, hist, cu, ci, ccu, cci,
                  tbuf, pbuf, outbuf, sh_u, sh_i, sh_t):
    cid = lax.axis_index("c")
    sid = lax.axis_index("s")
    row = cid * 8 + lax.div(sid, 2)
    half = lax.rem(sid, 2)
    lanes = _iota16()
    ones16 = jnp.ones((L,), jnp.int32)
    zero16 = jnp.zeros((L,), jnp.int32)

    pltpu.sync_copy(mask_hbm.at[row, pl.ds(half * HROW, HROW)], rbuf)

    def zh(i, _):
        hist[pl.ds(i * L, L)] = zero16
        return 0

    lax.fori_loop(0, 256, zh, 0, unroll=4)

    def s1(i, _):
        bits = rbuf[pl.ds(i * L, L)]
        r = bits ^ (MIN32 | lax.shift_right_arithmetic(bits, 31))
        rbuf[pl.ds(i * L, L)] = r
        b0 = lax.shift_right_logical(r, 24)
        plsc.addupdate_scatter(hist, [lanes * 256 + b0], ones16)
        return 0

    with jax.named_scope("ph_scan1"):
        lax.fori_loop(0, HROW // L, s1, 0, unroll=4)

    with jax.named_scope("ph_totals"):
        for g in range(16):
            tg = zero16
            for m in range(L):
                tg = tg + hist[pl.ds(m * 256 + g * L, L)]
            tbuf[pl.ds(g * L, L)] = tg
        pltpu.sync_copy(tbuf, sh_t.at[pl.ds(sid * 256, 256)])
    plsc.subcore_barrier()
    with jax.named_scope("ph_cross0"):
        pltpu.sync_copy(sh_t.at[pl.ds((sid ^ 1) * 256, 256)], pbuf)
        b0star, sstar0 = _crossing_from(
            lambda g: tbuf[pl.ds(g * L, L)] + pbuf[pl.ds(g * L, L)],
            jnp.int32(K))
    kr = K - sstar0

    gbase = half * HROW + lanes * SEG

    def s2(t, coffs):
        r = plsc.load_gather(rbuf, [lanes * SEG + t])
        cd = lax.shift_right_logical(r, 24) >= b0star
        plsc.store_scatter(cu, [coffs], r, mask=cd)
        plsc.store_scatter(ci, [coffs], gbase + t, mask=cd)
        return coffs + cd.astype(jnp.int32)

    with jax.named_scope("ph_scan2"):
        coffs = lax.fori_loop(0, SEG, s2, lanes * SEG, unroll=4)
    ccnts = coffs - lanes * SEG

    with jax.named_scope("ph_compact"):
        cptr = jnp.int32(0)
        for l in range(L):
            c_l = ccnts[l]

            def cpc(tt, _, l=l):
                vr = cu[pl.ds(l * SEG + tt * L, L)]
                vi = ci[pl.ds(l * SEG + tt * L, L)]
                dst = cptr + tt * L + lanes
                plsc.store_scatter(ccu, [dst], vr)
                plsc.store_scatter(cci, [dst], vi)
                return 0

            lax.fori_loop(0, (c_l + L - 1) // L, cpc, 0)
            cptr = cptr + c_l

        plsc.store_scatter(ccu, [cptr + lanes], zero16)
        cptr = pl.multiple_of((cptr + 7) & jnp.int32(-8), 8)

        @pl.when(half == 1)
        def _():
            pltpu.sync_copy(ccu.at[pl.ds(0, HROW)], sh_u.at[pl.ds(sid * HROW, HROW)])
            pltpu.sync_copy(cci.at[pl.ds(0, HROW)], sh_i.at[pl.ds(sid * HROW, HROW)])
            tbuf[pl.ds(0, L)] = ccnts
            pltpu.sync_copy(tbuf, sh_t.at[pl.ds(sid * 256, 256)])
    plsc.subcore_barrier()

    @pl.when(half == 0)
    def _():
        with jax.named_scope("ph_merge"):
            pltpu.sync_copy(sh_u.at[pl.ds((sid + 1) * HROW, HROW)], ccu.at[pl.ds(cptr, HROW)])
            pltpu.sync_copy(sh_i.at[pl.ds((sid + 1) * HROW, HROW)], cci.at[pl.ds(cptr, HROW)])
            pltpu.sync_copy(sh_t.at[pl.ds((sid + 1) * 256, 256)], pbuf)
            cn = cptr + jnp.sum(pbuf[pl.ds(0, L)])

        pfx = b0star
        kr2 = kr
        with jax.named_scope("ph_bytepasses"):
            for sh in (16, 8, 0):
                lax.fori_loop(0, 256, zh, 0, unroll=4)
                nv = (cn + L - 1) // L

                def sp(tt, _, sh=sh, pfx=pfx):
                    addr = tt * L + lanes
                    valid = addr < cn
                    r = ccu[pl.ds(tt * L, L)]
                    act = valid & (lax.shift_right_logical(r, sh + 8) == pfx)
                    b = lax.shift_right_logical(r, sh) & 255
                    plsc.addupdate_scatter(hist, [lanes * 256 + b], ones16,
                                           mask=act)
                    return 0

                lax.fori_loop(0, nv, sp, 0)

                def load_tg(g):
                    tg = zero16
                    for m in range(L):
                        tg = tg + hist[pl.ds(m * 256 + g * L, L)]
                    return tg

                bstar, sstar = _crossing_from(load_tg, kr2)
                kr2 = kr2 - sstar
                pfx = lax.shift_left(pfx, 8) | bstar
        tv = pfx
        tv_s = tv ^ MIN32

        nv = (cn + L - 1) // L

        def fs(tt, carry):
            ptr, neq = carry
            addr = tt * L + lanes
            valid = addr < cn
            r = ccu[pl.ds(tt * L, L)]
            iv = cci[pl.ds(tt * L, L)]
            rs = r ^ MIN32
            gt = valid & (rs > tv_s)
            eq = valid & (r == tv)
            eqc = eq.astype(jnp.int32)
            inc = plsc.cumsum(eqc)
            take = eq & ((inc - eqc + neq) < kr2)
            acc = gt | take
            acci = acc.astype(jnp.int32)
            inca = plsc.cumsum(acci)
            plsc.store_scatter(outbuf, [ptr + inca - acci], iv, mask=acc)
            return (ptr + jnp.sum(acci), neq + jnp.sum(eqc))

        with jax.named_scope("ph_select"):
            lax.fori_loop(0, nv, fs, (jnp.int32(0), jnp.int32(0)))
        pltpu.sync_copy(outbuf.at[pl.ds(0, K)], out_hbm.at[row])


_sc_topk = functools.partial(
    pl.kernel,
    out_type=jax.ShapeDtypeStruct((B, K), jnp.int32),
    mesh=plsc.VectorSubcoreMesh(core_axis_name="c", subcore_axis_name="s",
                                num_cores=2, num_subcores=16),
    compiler_params=pltpu.CompilerParams(needs_layout_passes=False),
    scratch_types=[
        pltpu.VMEM((HROW,), jnp.int32),
        pltpu.VMEM((256 * L,), jnp.int32),
        pltpu.VMEM((HROW,), jnp.int32),
        pltpu.VMEM((HROW,), jnp.int32),
        pltpu.VMEM((CCAP,), jnp.int32),
        pltpu.VMEM((CCAP,), jnp.int32),
        pltpu.VMEM((256,), jnp.int32),
        pltpu.VMEM((256,), jnp.int32),
        pltpu.VMEM((K + L,), jnp.int32),
        pltpu.VMEM_SHARED((16 * HROW,), jnp.int32),
        pltpu.VMEM_SHARED((16 * HROW,), jnp.int32),
        pltpu.VMEM_SHARED((16 * 256,), jnp.int32),
    ],
)(_sc_topk_body)


def _tc_body(idx_ref, idxc_ref, g_ref, nodes_ref, wg_ref, wa_ref, wb_ref,
             aw1_ref, ab1_ref, aw2_ref, ab2_ref, ab0_ref,
             cw0_ref, cb0_ref, cw1_ref, cb1_ref, cw2_ref, cb2_ref,
             pi_ref, val_ref):
    idxv = idx_ref[0]
    i_sel = lax.shift_right_logical(idxv, 7)
    r_i = lax.broadcasted_iota(jnp.int32, (N, N), 0)
    c_i = lax.broadcasted_iota(jnp.int32, (N, N), 1)
    ohit = (r_i == i_sel).astype(jnp.float32)

    idxc = idxc_ref[0]
    i_col = lax.shift_right_logical(idxc, 7)
    j_col = idxc & 127
    ohj = (c_i == j_col).astype(jnp.float32)
    ohi = (c_i == i_col).astype(jnp.float32)

    nodes = nodes_ref[0]
    g = g_ref[0]
    a = jnp.dot(nodes, wa_ref[...], preferred_element_type=jnp.float32)
    bm = jnp.dot(nodes, wb_ref[...], preferred_element_type=jnp.float32)
    c = jnp.dot(g, wg_ref[...], preferred_element_type=jnp.float32) + ab0_ref[...]

    hp = jax.lax.Precision.HIGHEST
    a_sel = jnp.dot(ohj, a, preferred_element_type=jnp.float32, precision=hp)
    b_sel = jnp.dot(ohi, bm, preferred_element_type=jnp.float32, precision=hp)
    h0 = jax.nn.relu(a_sel + b_sel + c)
    h1 = jax.nn.relu(jnp.dot(h0, aw1_ref[...], preferred_element_type=jnp.float32)
                     + ab1_ref[...])
    lg = jnp.dot(h1, aw2_ref[...], preferred_element_type=jnp.float32) + ab2_ref[...]

    mx = jnp.max(lg)
    e = jnp.exp(lg - mx)
    pi_col = e / jnp.sum(e)

    ohj_scaled = ohj * pi_col
    pi_ref[0] = jnp.dot(ohit, ohj_scaled, preferred_element_type=jnp.float32,
                        precision=hp)

    hv = jax.nn.relu(jnp.dot(g, cw0_ref[...], preferred_element_type=jnp.float32)
                     + cb0_ref[...])
    hv = jax.nn.relu(jnp.dot(hv, cw1_ref[...], preferred_element_type=jnp.float32)
                     + cb1_ref[...])
    val_ref[0] = jnp.dot(hv, cw2_ref[...], preferred_element_type=jnp.float32) + cb2_ref[...]


@jax.jit
def kernel(embedded_features, aW0, ab0, aW1, ab1, aW2, ab2,
           cW0, cb0, cW1, cb1, cW2, cb2):
    gan = embedded_features[:, :, :HID]
    g3 = gan[:, :1, :]
    nodes = gan[:, 1:, :]
    maskrow = embedded_features[:, 1:, HID:].reshape(B, ROW_N)
    maskbits = lax.bitcast_convert_type(maskrow, jnp.int32)

    idx = _sc_topk(maskbits)
    idx3 = idx.reshape(B, 1, K)
    idxc3 = idx.reshape(B, K, 1)

    wg = aW0[:HID]
    wa = aW0[HID:2 * HID]
    wb = aW0[2 * HID:]
    ab0r = ab0.reshape(1, HID)
    ab1r = ab1.reshape(1, HID)
    ab2r = ab2.reshape(1, 1)
    cb0r = cb0.reshape(1, HID)
    cb1r = cb1.reshape(1, HID)
    cb2r = cb2.reshape(1, 1)

    full = lambda shape: pl.BlockSpec(shape, lambda i: (0,) * len(shape))
    per_b3 = lambda s1, s2: pl.BlockSpec((1, s1, s2), lambda i: (i, 0, 0))

    pi, value = pl.pallas_call(
        _tc_body,
        grid=(B,),
        in_specs=[
            per_b3(1, K),
            per_b3(K, 1),
            per_b3(1, HID),
            per_b3(N, HID),
            full((HID, HID)),
            full((HID, HID)),
            full((HID, HID)),
            full((HID, HID)),
            full((1, HID)),
            full((HID, 1)),
            full((1, 1)),
            full((1, HID)),
            full((HID, HID)),
            full((1, HID)),
            full((HID, HID)),
            full((1, HID)),
            full((HID, 1)),
            full((1, 1)),
        ],
        out_specs=[per_b3(N, N), per_b3(1, 1)],
        out_shape=[
            jax.ShapeDtypeStruct((B, N, N), jnp.float32),
            jax.ShapeDtypeStruct((B, 1, 1), jnp.float32),
        ],
    )(idx3, idxc3, g3, nodes, wg, wa, wb, aW1, ab1r, aW2, ab2r, ab0r,
      cW0, cb0r, cW1, cb1r, cW2, cb2r)

    return pi.reshape(B, N * N), value


# speedup vs baseline: 4.8150x; 1.1453x over previous
import functools

import numpy as np

import jax
import jax.numpy as jnp
from jax import lax
from jax.experimental import pallas as pl
from jax.experimental.pallas import tpu as pltpu
from jax.experimental.pallas import tpu_sc as plsc

B = 16
N = 128
HID = 64
K = 128
ROW_N = N * N
L = 16
HROW = ROW_N // 2
SEG = HROW // L
CCAP = ROW_N + L
MIN32 = np.int32(-2147483648)


def _iota16():
    return lax.broadcasted_iota(jnp.int32, (L,), 0)


def _crossing_from(load_tg, kr):
    lanes = _iota16()
    tgs = []
    sums = []
    for g in range(16):
        tg = load_tg(g)
        tgs.append(tg)
        sums.append(jnp.sum(tg))
    bstar = jnp.int32(0)
    sstar = jnp.int32(0)
    above = jnp.int32(0)
    for g in range(15, -1, -1):
        tg = tgs[g]
        sfx_incl = lax.rev(plsc.cumsum(lax.rev(tg, (0,))), (0,))
        s_b = above + sfx_incl - tg
        okv = (s_b < kr) & (s_b + tg >= kr)
        bstar = bstar + jnp.sum(jnp.where(okv, g * L + lanes, 0))
        sstar = sstar + jnp.sum(jnp.where(okv, s_b, 0))
        above = above + sums[g]
    return bstar, sstar


def _sc_topk_body(mask_hbm, out_hbm, rbuf, hist, cu, ci, ccu, cci,
                  tbuf, pbuf, outbuf, sh_u, sh_i, sh_t):
    cid = lax.axis_index("c")
    sid = lax.axis_index("s")
    row = cid * 8 + lax.div(sid, 2)
    half = lax.rem(sid, 2)
    lanes = _iota16()
    ones16 = jnp.ones((L,), jnp.int32)
    zero16 = jnp.zeros((L,), jnp.int32)

    pltpu.sync_copy(mask_hbm.at[row, pl.ds(half * HROW, HROW)], rbuf)

    def zh(i, _):
        hist[pl.ds(i * L, L)] = zero16
        return 0

    lax.fori_loop(0, 256, zh, 0, unroll=4)

    def s1(i, _):
        bits = rbuf[pl.ds(i * L, L)]
        r = bits ^ (MIN32 | lax.shift_right_arithmetic(bits, 31))
        rbuf[pl.ds(i * L, L)] = r
        b0 = lax.shift_right_logical(r, 24)
        plsc.addupdate_scatter(hist, [lanes * 256 + b0], ones16)
        return 0

    with jax.named_scope("ph_scan1"):
        lax.fori_loop(0, HROW // L, s1, 0, unroll=8)

    with jax.named_scope("ph_totals"):
        for g in range(16):
            tg = zero16
            for m in range(L):
                tg = tg + hist[pl.ds(m * 256 + g * L, L)]
            tbuf[pl.ds(g * L, L)] = tg
        pltpu.sync_copy(tbuf, sh_t.at[pl.ds(sid * 256, 256)])
    plsc.subcore_barrier()
    with jax.named_scope("ph_cross0"):
        pltpu.sync_copy(sh_t.at[pl.ds((sid ^ 1) * 256, 256)], pbuf)
        b0star, sstar0 = _crossing_from(
            lambda g: tbuf[pl.ds(g * L, L)] + pbuf[pl.ds(g * L, L)],
            jnp.int32(K))
    kr = K - sstar0

    gbase = half * HROW + lanes * SEG

    def s2(t, coffs):
        r = plsc.load_gather(rbuf, [lanes * SEG + t])
        cd = lax.shift_right_logical(r, 24) >= b0star
        plsc.store_scatter(cu, [coffs], r, mask=cd)
        plsc.store_scatter(ci, [coffs], gbase + t, mask=cd)
        return coffs + cd.astype(jnp.int32)

    with jax.named_scope("ph_scan2"):
        coffs = lax.fori_loop(0, SEG, s2, lanes * SEG, unroll=8)
    ccnts = coffs - lanes * SEG

    with jax.named_scope("ph_compact"):
        cptr = jnp.int32(0)
        for l in range(L):
            c_l = ccnts[l]

            def cpc(tt, _, l=l):
                vr = cu[pl.ds(l * SEG + tt * L, L)]
                vi = ci[pl.ds(l * SEG + tt * L, L)]
                dst = cptr + tt * L + lanes
                plsc.store_scatter(ccu, [dst], vr)
                plsc.store_scatter(cci, [dst], vi)
                return 0

            lax.fori_loop(0, (c_l + L - 1) // L, cpc, 0)
            cptr = cptr + c_l

        plsc.store_scatter(ccu, [cptr + lanes], zero16)
        cptr = pl.multiple_of((cptr + 7) & jnp.int32(-8), 8)

        @pl.when(half == 1)
        def _():
            pltpu.sync_copy(ccu.at[pl.ds(0, HROW)], sh_u.at[pl.ds(sid * HROW, HROW)])
            pltpu.sync_copy(cci.at[pl.ds(0, HROW)], sh_i.at[pl.ds(sid * HROW, HROW)])
            tbuf[pl.ds(0, L)] = ccnts
            pltpu.sync_copy(tbuf, sh_t.at[pl.ds(sid * 256, 256)])
    plsc.subcore_barrier()

    @pl.when(half == 0)
    def _():
        with jax.named_scope("ph_merge"):
            pltpu.sync_copy(sh_u.at[pl.ds((sid + 1) * HROW, HROW)], ccu.at[pl.ds(cptr, HROW)])
            pltpu.sync_copy(sh_i.at[pl.ds((sid + 1) * HROW, HROW)], cci.at[pl.ds(cptr, HROW)])
            pltpu.sync_copy(sh_t.at[pl.ds((sid + 1) * 256, 256)], pbuf)
            cn = cptr + jnp.sum(pbuf[pl.ds(0, L)])

        pfx = b0star
        kr2 = kr
        with jax.named_scope("ph_bytepasses"):
            for sh in (16, 8, 0):
                lax.fori_loop(0, 256, zh, 0, unroll=4)
                nv = (cn + L - 1) // L

                def sp(tt, _, sh=sh, pfx=pfx):
                    addr = tt * L + lanes
                    valid = addr < cn
                    r = ccu[pl.ds(tt * L, L)]
                    act = valid & (lax.shift_right_logical(r, sh + 8) == pfx)
                    b = lax.shift_right_logical(r, sh) & 255
                    plsc.addupdate_scatter(hist, [lanes * 256 + b], ones16,
                                           mask=act)
                    return 0

                lax.fori_loop(0, nv, sp, 0)

                def load_tg(g):
                    tg = zero16
                    for m in range(L):
                        tg = tg + hist[pl.ds(m * 256 + g * L, L)]
                    return tg

                bstar, sstar = _crossing_from(load_tg, kr2)
                kr2 = kr2 - sstar
                pfx = lax.shift_left(pfx, 8) | bstar
        tv = pfx
        tv_s = tv ^ MIN32

        nv = (cn + L - 1) // L

        def fs(tt, carry):
            ptr, neq = carry
            addr = tt * L + lanes
            valid = addr < cn
            r = ccu[pl.ds(tt * L, L)]
            iv = cci[pl.ds(tt * L, L)]
            rs = r ^ MIN32
            gt = valid & (rs > tv_s)
            eq = valid & (r == tv)
            eqc = eq.astype(jnp.int32)
            inc = plsc.cumsum(eqc)
            take = eq & ((inc - eqc + neq) < kr2)
            acc = gt | take
            acci = acc.astype(jnp.int32)
            inca = plsc.cumsum(acci)
            plsc.store_scatter(outbuf, [ptr + inca - acci], iv, mask=acc)
            return (ptr + jnp.sum(acci), neq + jnp.sum(eqc))

        with jax.named_scope("ph_select"):
            lax.fori_loop(0, nv, fs, (jnp.int32(0), jnp.int32(0)))
        pltpu.sync_copy(outbuf.at[pl.ds(0, K)], out_hbm.at[row])


_sc_topk = functools.partial(
    pl.kernel,
    out_type=jax.ShapeDtypeStruct((B, K), jnp.int32),
    mesh=plsc.VectorSubcoreMesh(core_axis_name="c", subcore_axis_name="s",
                                num_cores=2, num_subcores=16),
    compiler_params=pltpu.CompilerParams(needs_layout_passes=False),
    scratch_types=[
        pltpu.VMEM((HROW,), jnp.int32),
        pltpu.VMEM((256 * L,), jnp.int32),
        pltpu.VMEM((HROW,), jnp.int32),
        pltpu.VMEM((HROW,), jnp.int32),
        pltpu.VMEM((CCAP,), jnp.int32),
        pltpu.VMEM((CCAP,), jnp.int32),
        pltpu.VMEM((256,), jnp.int32),
        pltpu.VMEM((256,), jnp.int32),
        pltpu.VMEM((K + L,), jnp.int32),
        pltpu.VMEM_SHARED((16 * HROW,), jnp.int32),
        pltpu.VMEM_SHARED((16 * HROW,), jnp.int32),
        pltpu.VMEM_SHARED((16 * 256,), jnp.int32),
    ],
)(_sc_topk_body)


TCB = 4


def _tc_body(idx_ref, g_ref, nodes_ref, wg_ref, wa_ref, wb_ref,
             aw1_ref, ab1_ref, aw2_ref, ab2_ref, ab0_ref,
             cw0_ref, cb0_ref, cw1_ref, cb1_ref, cw2_ref, cb2_ref,
             pi_ref, val_ref):
    hp = jax.lax.Precision.HIGHEST
    dn_c0 = (((0,), (0,)), ((), ()))
    dn_c11 = (((1,), (1,)), ((), ()))
    dn_mv = (((0,), (1,)), ((), ()))
    r_i = lax.broadcasted_iota(jnp.int32, (N, N), 0)

    for bb in range(TCB):
        idxv = idx_ref[bb]
        i_sel = lax.shift_right_logical(idxv, 7)
        j_sel = idxv & 127
        ohit = (r_i == i_sel).astype(jnp.float32)
        ohjt = (r_i == j_sel).astype(jnp.float32)

        nodes = nodes_ref[bb]
        g = g_ref[bb]
        a = jnp.dot(nodes, wa_ref[...], preferred_element_type=jnp.float32)
        bm = jnp.dot(nodes, wb_ref[...], preferred_element_type=jnp.float32)
        c = jnp.dot(g, wg_ref[...], preferred_element_type=jnp.float32) + ab0_ref[...]

        a_sel = lax.dot_general(ohjt, a, dn_c0,
                                preferred_element_type=jnp.float32, precision=hp)
        b_sel = lax.dot_general(ohit, bm, dn_c0,
                                preferred_element_type=jnp.float32, precision=hp)
        h0 = jax.nn.relu(a_sel + b_sel + c)
        h1 = jax.nn.relu(jnp.dot(h0, aw1_ref[...],
                                 preferred_element_type=jnp.float32)
                         + ab1_ref[...])
        lg_row = lax.dot_general(aw2_ref[...], h1, dn_mv,
                                 preferred_element_type=jnp.float32) + ab2_ref[...]

        mx = jnp.max(lg_row)
        e = jnp.exp(lg_row - mx)
        pi_row = e / jnp.sum(e)

        m1t = ohit * pi_row
        pi_ref[bb] = lax.dot_general(m1t, ohjt, dn_c11,
                                     preferred_element_type=jnp.float32,
                                     precision=hp)

        hv = jax.nn.relu(jnp.dot(g, cw0_ref[...],
                                 preferred_element_type=jnp.float32)
                         + cb0_ref[...])
        hv = jax.nn.relu(jnp.dot(hv, cw1_ref[...],
                                 preferred_element_type=jnp.float32)
                         + cb1_ref[...])
        val_ref[bb] = jnp.dot(hv, cw2_ref[...],
                              preferred_element_type=jnp.float32) + cb2_ref[...]


@jax.jit
def kernel(embedded_features, aW0, ab0, aW1, ab1, aW2, ab2,
           cW0, cb0, cW1, cb1, cW2, cb2):
    gan = embedded_features[:, :, :HID]
    g3 = gan[:, :1, :]
    nodes = gan[:, 1:, :]
    maskrow = embedded_features[:, 1:, HID:].reshape(B, ROW_N)
    maskbits = lax.bitcast_convert_type(maskrow, jnp.int32)

    idx = _sc_topk(maskbits)
    idx3 = idx.reshape(B, 1, K)

    wg = aW0[:HID]
    wa = aW0[HID:2 * HID]
    wb = aW0[2 * HID:]
    ab0r = ab0.reshape(1, HID)
    ab1r = ab1.reshape(1, HID)
    ab2r = ab2.reshape(1, 1)
    cb0r = cb0.reshape(1, HID)
    cb1r = cb1.reshape(1, HID)
    cb2r = cb2.reshape(1, 1)

    full = lambda shape: pl.BlockSpec(shape, lambda i: (0,) * len(shape))
    per_b3 = lambda s1, s2: pl.BlockSpec((TCB, s1, s2), lambda i: (i, 0, 0))

    pi, value = pl.pallas_call(
        _tc_body,
        grid=(B // TCB,),
        in_specs=[
            per_b3(1, K),
            per_b3(1, HID),
            per_b3(N, HID),
            full((HID, HID)),
            full((HID, HID)),
            full((HID, HID)),
            full((HID, HID)),
            full((1, HID)),
            full((HID, 1)),
            full((1, 1)),
            full((1, HID)),
            full((HID, HID)),
            full((1, HID)),
            full((HID, HID)),
            full((1, HID)),
            full((HID, 1)),
            full((1, 1)),
        ],
        out_specs=[per_b3(N, N), per_b3(1, 1)],
        out_shape=[
            jax.ShapeDtypeStruct((B, N, N), jnp.float32),
            jax.ShapeDtypeStruct((B, 1, 1), jnp.float32),
        ],
    )(idx3, g3, nodes, wg, wa, wb, aW1, ab1r, aW2, ab2r, ab0r,
      cW0, cb0r, cW1, cb1r, cW2, cb2r)

    return pi.reshape(B, N * N), value


# speedup vs baseline: 4.8641x; 1.0102x over previous
import functools

import numpy as np

import jax
import jax.numpy as jnp
from jax import lax
from jax.experimental import pallas as pl
from jax.experimental.pallas import tpu as pltpu
from jax.experimental.pallas import tpu_sc as plsc

B = 16
N = 128
HID = 64
K = 128
ROW_N = N * N
L = 16
HROW = ROW_N // 2
SEG = HROW // L
CCAP = ROW_N + L
MIN32 = np.int32(-2147483648)


def _iota16():
    return lax.broadcasted_iota(jnp.int32, (L,), 0)


def _crossing_from(load_tg, kr):
    lanes = _iota16()
    tgs = []
    sums = []
    for g in range(16):
        tg = load_tg(g)
        tgs.append(tg)
        sums.append(jnp.sum(tg))
    bstar = jnp.int32(0)
    sstar = jnp.int32(0)
    above = jnp.int32(0)
    for g in range(15, -1, -1):
        tg = tgs[g]
        sfx_incl = lax.rev(plsc.cumsum(lax.rev(tg, (0,))), (0,))
        s_b = above + sfx_incl - tg
        okv = (s_b < kr) & (s_b + tg >= kr)
        bstar = bstar + jnp.sum(jnp.where(okv, g * L + lanes, 0))
        sstar = sstar + jnp.sum(jnp.where(okv, s_b, 0))
        above = above + sums[g]
    return bstar, sstar


def _sc_topk_body(mask_hbm, out_hbm, rbuf, hist, cu, ci, ccu, cci,
                  tbuf, pbuf, outbuf, sh_u, sh_i, sh_t):
    cid = lax.axis_index("c")
    sid = lax.axis_index("s")
    row = cid * 8 + lax.div(sid, 2)
    half = lax.rem(sid, 2)
    lanes = _iota16()
    ones16 = jnp.ones((L,), jnp.int32)
    zero16 = jnp.zeros((L,), jnp.int32)

    pltpu.sync_copy(mask_hbm.at[row, pl.ds(half * HROW, HROW)], rbuf)

    def zh(i, _):
        hist[pl.ds(i * L, L)] = zero16
        return 0

    lax.fori_loop(0, 256, zh, 0, unroll=4)

    def s1(i, _):
        bits = rbuf[pl.ds(i * L, L)]
        r = bits ^ (MIN32 | lax.shift_right_arithmetic(bits, 31))
        rbuf[pl.ds(i * L, L)] = r
        b0 = lax.shift_right_logical(r, 24)
        plsc.addupdate_scatter(hist, [lanes * 256 + b0], ones16)
        return 0

    lax.fori_loop(0, HROW // L, s1, 0, unroll=8)

    for g in range(16):
        tg = zero16
        for m in range(L):
            tg = tg + hist[pl.ds(m * 256 + g * L, L)]
        tbuf[pl.ds(g * L, L)] = tg
    pltpu.sync_copy(tbuf, sh_t.at[pl.ds(sid * 256, 256)])
    plsc.subcore_barrier()
    pltpu.sync_copy(sh_t.at[pl.ds((sid ^ 1) * 256, 256)], pbuf)
    b0star, sstar0 = _crossing_from(
        lambda g: tbuf[pl.ds(g * L, L)] + pbuf[pl.ds(g * L, L)],
        jnp.int32(K))
    kr = K - sstar0

    gbase = half * HROW + lanes * SEG

    def s2(t, coffs):
        r = plsc.load_gather(rbuf, [lanes * SEG + t])
        cd = lax.shift_right_logical(r, 24) >= b0star
        plsc.store_scatter(cu, [coffs], r, mask=cd)
        plsc.store_scatter(ci, [coffs], gbase + t, mask=cd)
        return coffs + cd.astype(jnp.int32)

    coffs = lax.fori_loop(0, SEG, s2, lanes * SEG, unroll=8)
    ccnts = coffs - lanes * SEG

    if True:
        cptr = jnp.int32(0)
        for l in range(L):
            c_l = ccnts[l]

            def cpc(tt, _, l=l):
                vr = cu[pl.ds(l * SEG + tt * L, L)]
                vi = ci[pl.ds(l * SEG + tt * L, L)]
                dst = cptr + tt * L + lanes
                plsc.store_scatter(ccu, [dst], vr)
                plsc.store_scatter(cci, [dst], vi)
                return 0

            lax.fori_loop(0, (c_l + L - 1) // L, cpc, 0)
            cptr = cptr + c_l

        plsc.store_scatter(ccu, [cptr + lanes], zero16)
        cptr = pl.multiple_of((cptr + 7) & jnp.int32(-8), 8)

        @pl.when(half == 1)
        def _():
            pltpu.sync_copy(ccu.at[pl.ds(0, HROW)], sh_u.at[pl.ds(sid * HROW, HROW)])
            pltpu.sync_copy(cci.at[pl.ds(0, HROW)], sh_i.at[pl.ds(sid * HROW, HROW)])
            tbuf[pl.ds(0, L)] = ccnts
            pltpu.sync_copy(tbuf, sh_t.at[pl.ds(sid * 256, 256)])
    plsc.subcore_barrier()

    @pl.when(half == 0)
    def _():
        if True:
            pltpu.sync_copy(sh_u.at[pl.ds((sid + 1) * HROW, HROW)], ccu.at[pl.ds(cptr, HROW)])
            pltpu.sync_copy(sh_i.at[pl.ds((sid + 1) * HROW, HROW)], cci.at[pl.ds(cptr, HROW)])
            pltpu.sync_copy(sh_t.at[pl.ds((sid + 1) * 256, 256)], pbuf)
            cn = cptr + jnp.sum(pbuf[pl.ds(0, L)])

        pfx = b0star
        kr2 = kr
        if True:
            for sh in (16, 8, 0):
                lax.fori_loop(0, 256, zh, 0, unroll=4)
                nv = (cn + L - 1) // L

                def sp(tt, _, sh=sh, pfx=pfx):
                    addr = tt * L + lanes
                    valid = addr < cn
                    r = ccu[pl.ds(tt * L, L)]
                    act = valid & (lax.shift_right_logical(r, sh + 8) == pfx)
                    b = lax.shift_right_logical(r, sh) & 255
                    plsc.addupdate_scatter(hist, [lanes * 256 + b], ones16,
                                           mask=act)
                    return 0

                lax.fori_loop(0, nv, sp, 0)

                def load_tg(g):
                    tg = zero16
                    for m in range(L):
                        tg = tg + hist[pl.ds(m * 256 + g * L, L)]
                    return tg

                bstar, sstar = _crossing_from(load_tg, kr2)
                kr2 = kr2 - sstar
                pfx = lax.shift_left(pfx, 8) | bstar
        tv = pfx
        tv_s = tv ^ MIN32

        nv = (cn + L - 1) // L

        def fs(tt, carry):
            ptr, neq = carry
            addr = tt * L + lanes
            valid = addr < cn
            r = ccu[pl.ds(tt * L, L)]
            iv = cci[pl.ds(tt * L, L)]
            rs = r ^ MIN32
            gt = valid & (rs > tv_s)
            eq = valid & (r == tv)
            eqc = eq.astype(jnp.int32)
            inc = plsc.cumsum(eqc)
            take = eq & ((inc - eqc + neq) < kr2)
            acc = gt | take
            acci = acc.astype(jnp.int32)
            inca = plsc.cumsum(acci)
            plsc.store_scatter(outbuf, [ptr + inca - acci], iv, mask=acc)
            return (ptr + jnp.sum(acci), neq + jnp.sum(eqc))

        lax.fori_loop(0, nv, fs, (jnp.int32(0), jnp.int32(0)))
        pltpu.sync_copy(outbuf.at[pl.ds(0, K)], out_hbm.at[row])


_sc_topk = functools.partial(
    pl.kernel,
    out_type=jax.ShapeDtypeStruct((B, K), jnp.int32),
    mesh=plsc.VectorSubcoreMesh(core_axis_name="c", subcore_axis_name="s",
                                num_cores=2, num_subcores=16),
    compiler_params=pltpu.CompilerParams(needs_layout_passes=False),
    scratch_types=[
        pltpu.VMEM((HROW,), jnp.int32),
        pltpu.VMEM((256 * L,), jnp.int32),
        pltpu.VMEM((HROW,), jnp.int32),
        pltpu.VMEM((HROW,), jnp.int32),
        pltpu.VMEM((CCAP,), jnp.int32),
        pltpu.VMEM((CCAP,), jnp.int32),
        pltpu.VMEM((256,), jnp.int32),
        pltpu.VMEM((256,), jnp.int32),
        pltpu.VMEM((K + L,), jnp.int32),
        pltpu.VMEM_SHARED((16 * HROW,), jnp.int32),
        pltpu.VMEM_SHARED((16 * HROW,), jnp.int32),
        pltpu.VMEM_SHARED((16 * 256,), jnp.int32),
    ],
)(_sc_topk_body)


TCB = 8


def _tc_body(idx_ref, g_ref, nodes_ref, wg_ref, wa_ref, wb_ref,
             aw1_ref, ab1_ref, aw2_ref, ab2_ref, ab0_ref,
             cw0_ref, cb0_ref, cw1_ref, cb1_ref, cw2_ref, cb2_ref,
             pi_ref, val_ref):
    hp = jax.lax.Precision.HIGHEST
    dn_c0 = (((0,), (0,)), ((), ()))
    dn_c11 = (((1,), (1,)), ((), ()))
    dn_mv = (((0,), (1,)), ((), ()))
    r_i = lax.broadcasted_iota(jnp.int32, (N, N), 0)

    for bb in range(TCB):
        idxv = idx_ref[bb]
        i_sel = lax.shift_right_logical(idxv, 7)
        j_sel = idxv & 127
        ohit = (r_i == i_sel).astype(jnp.float32)
        ohjt = (r_i == j_sel).astype(jnp.float32)

        nodes = nodes_ref[bb]
        g = g_ref[bb]
        a = jnp.dot(nodes, wa_ref[...], preferred_element_type=jnp.float32)
        bm = jnp.dot(nodes, wb_ref[...], preferred_element_type=jnp.float32)
        c = jnp.dot(g, wg_ref[...], preferred_element_type=jnp.float32) + ab0_ref[...]

        a_sel = lax.dot_general(ohjt, a, dn_c0,
                                preferred_element_type=jnp.float32, precision=hp)
        b_sel = lax.dot_general(ohit, bm, dn_c0,
                                preferred_element_type=jnp.float32, precision=hp)
        h0 = jax.nn.relu(a_sel + b_sel + c)
        h1 = jax.nn.relu(jnp.dot(h0, aw1_ref[...],
                                 preferred_element_type=jnp.float32)
                         + ab1_ref[...])
        lg_row = lax.dot_general(aw2_ref[...], h1, dn_mv,
                                 preferred_element_type=jnp.float32) + ab2_ref[...]

        mx = jnp.max(lg_row)
        e = jnp.exp(lg_row - mx)
        pi_row = e / jnp.sum(e)

        m1t = ohit * pi_row
        pi_ref[bb] = lax.dot_general(m1t, ohjt, dn_c11,
                                     preferred_element_type=jnp.float32,
                                     precision=hp)

        hv = jax.nn.relu(jnp.dot(g, cw0_ref[...],
                                 preferred_element_type=jnp.float32)
                         + cb0_ref[...])
        hv = jax.nn.relu(jnp.dot(hv, cw1_ref[...],
                                 preferred_element_type=jnp.float32)
                         + cb1_ref[...])
        val_ref[bb] = jnp.dot(hv, cw2_ref[...],
                              preferred_element_type=jnp.float32) + cb2_ref[...]


@jax.jit
def kernel(embedded_features, aW0, ab0, aW1, ab1, aW2, ab2,
           cW0, cb0, cW1, cb1, cW2, cb2):
    gan = embedded_features[:, :, :HID]
    g3 = gan[:, :1, :]
    nodes = gan[:, 1:, :]
    maskrow = embedded_features[:, 1:, HID:].reshape(B, ROW_N)
    maskbits = lax.bitcast_convert_type(maskrow, jnp.int32)

    idx = _sc_topk(maskbits)
    idx3 = idx.reshape(B, 1, K)

    wg = aW0[:HID]
    wa = aW0[HID:2 * HID]
    wb = aW0[2 * HID:]
    ab0r = ab0.reshape(1, HID)
    ab1r = ab1.reshape(1, HID)
    ab2r = ab2.reshape(1, 1)
    cb0r = cb0.reshape(1, HID)
    cb1r = cb1.reshape(1, HID)
    cb2r = cb2.reshape(1, 1)

    full = lambda shape: pl.BlockSpec(shape, lambda i: (0,) * len(shape))
    per_b3 = lambda s1, s2: pl.BlockSpec((TCB, s1, s2), lambda i: (i, 0, 0))

    pi, value = pl.pallas_call(
        _tc_body,
        grid=(B // TCB,),
        in_specs=[
            per_b3(1, K),
            per_b3(1, HID),
            per_b3(N, HID),
            full((HID, HID)),
            full((HID, HID)),
            full((HID, HID)),
            full((HID, HID)),
            full((1, HID)),
            full((HID, 1)),
            full((1, 1)),
            full((1, HID)),
            full((HID, HID)),
            full((1, HID)),
            full((HID, HID)),
            full((1, HID)),
            full((HID, 1)),
            full((1, 1)),
        ],
        out_specs=[per_b3(N, N), per_b3(1, 1)],
        out_shape=[
            jax.ShapeDtypeStruct((B, N, N), jnp.float32),
            jax.ShapeDtypeStruct((B, 1, 1), jnp.float32),
        ],
    )(idx3, g3, nodes, wg, wa, wb, aW1, ab1r, aW2, ab2r, ab0r,
      cW0, cb0r, cW1, cb1r, cW2, cb2r)

    return pi.reshape(B, N * N), value
